# baseline (speedup 1.0000x reference)
import functools

import jax
import jax.numpy as jnp
from jax import lax
from jax.experimental import pallas as pl
from jax.experimental.pallas import tpu as pltpu
from jax.experimental.pallas import tpu_sc as plsc

_E = 8
_K = 2
_BM = 128
_FFC = 512
_L = 16
_NC = 2
_NS = 16
_NW = _NC * _NS
_PAD = 8


def _logits_body(x_ref, gw_ref, out_ref):
    out_ref[...] = lax.dot_general(
        x_ref[...], gw_ref[...], (((1,), (1,)), ((), ())),
        preferred_element_type=jnp.float32)


def _router_logits(x, gate_w):
    T, D = x.shape
    bt = 1024
    return pl.pallas_call(
        _logits_body,
        grid=(T // bt,),
        in_specs=[pl.BlockSpec((bt, D), lambda i: (i, 0)),
                  pl.BlockSpec((_E, D), lambda i: (0, 0))],
        out_specs=pl.BlockSpec((bt, _E), lambda i: (i, 0)),
        out_shape=jax.ShapeDtypeStruct((T, _E), jnp.float32),
    )(x, gate_w)


def _make_route_dispatch(T, D):
    TPW = T // _NW
    PG = TPW // _L
    N = T * _K + _E * _BM
    G = N // _BM
    GP = ((G + _L - 1) // _L) * _L
    XCH = 64

    mesh = plsc.VectorSubcoreMesh(core_axis_name="c", subcore_axis_name="s")

    @functools.partial(
        pl.kernel,
        out_type=[
            jax.ShapeDtypeStruct((N, D), jnp.float32),
            jax.ShapeDtypeStruct((T,), jnp.int32),
            jax.ShapeDtypeStruct((T,), jnp.int32),
            jax.ShapeDtypeStruct((T,), jnp.float32),
            jax.ShapeDtypeStruct((T,), jnp.float32),
            jax.ShapeDtypeStruct((GP,), jnp.int32),
        ],
        mesh=mesh,
        scratch_types=[
            pltpu.VMEM((TPW, _E), jnp.float32),
            pltpu.VMEM((TPW,), jnp.int32),
            pltpu.VMEM((TPW,), jnp.int32),
            pltpu.VMEM((TPW,), jnp.float32),
            pltpu.VMEM((TPW,), jnp.float32),
            pltpu.VMEM((2 * TPW,), jnp.int32),
            pltpu.VMEM((_L,), jnp.int32),
            pltpu.VMEM((_NW + _PAD, _L), jnp.int32),
            pltpu.VMEM((XCH, D), jnp.float32),
            pltpu.VMEM((XCH,), jnp.int32),
            pltpu.VMEM((XCH,), jnp.int32),
            pltpu.VMEM((GP,), jnp.int32),
            pltpu.SMEM((_E,), jnp.int32),
            pltpu.VMEM_SHARED((_NW + _PAD, _L), jnp.int32),
            pltpu.SemaphoreType.DMA,
            pltpu.SemaphoreType.DMA,
        ],
        compiler_params=pltpu.CompilerParams(needs_layout_passes=False),
    )
    def route(logits_hbm, x_hbm, xs_hbm, s1_hbm, s2_hbm, w1_hbm, w2_hbm,
              be_hbm, lgbuf, i1v, i2v, w1v, w2v, slots, cstage, cgridv,
              xbuf, idx1, idx2, bev, run, cgrid_sp, sem1, sem2):
        c = lax.axis_index("c")
        s = lax.axis_index("s")
        jown = c * _NS + s
        jalt = (1 - c) * _NS + s
        iota = lax.iota(jnp.int32, _L)

        def top2_counts(store):
            cnt = jnp.zeros((_L,), jnp.int32)
            for g in range(PG):
                rows = g * _L + iota
                vals = [
                    plsc.load_gather(
                        lgbuf, [rows, jnp.full((_L,), e, jnp.int32)])
                    for e in range(_E)
                ]
                m1 = vals[0]
                e1 = jnp.zeros((_L,), jnp.int32)
                for e in range(1, _E):
                    gt = vals[e] > m1
                    m1 = jnp.where(gt, vals[e], m1)
                    e1 = jnp.where(gt, e, e1)
                m2 = jnp.full((_L,), -jnp.inf, jnp.float32)
                e2 = jnp.zeros((_L,), jnp.int32)
                for e in range(_E):
                    ve = jnp.where(e1 == e, -jnp.inf, vals[e])
                    gt = ve > m2
                    m2 = jnp.where(gt, ve, m2)
                    e2 = jnp.where(gt, e, e2)
                for e in range(_E):
                    pc = (plsc.all_reduce_population_count(e1 == e)
                          + plsc.all_reduce_population_count(e2 == e))
                    cnt = jnp.where(iota == e, cnt + pc, cnt)
                if store:
                    sl = pl.ds(g * _L, _L)
                    i1v[sl] = e1
                    i2v[sl] = e2
                    ww1 = 1.0 / (1.0 + jnp.exp(m2 - m1))
                    w1v[sl] = ww1
                    w2v[sl] = 1.0 - ww1
            return cnt

        pltpu.sync_copy(logits_hbm.at[pl.ds(jalt * TPW, TPW)], lgbuf)
        cstage[...] = top2_counts(False)
        pltpu.sync_copy(cstage, cgrid_sp.at[jalt + _PAD])
        pltpu.sync_copy(logits_hbm.at[pl.ds(jown * TPW, TPW)], lgbuf)
        cstage[...] = top2_counts(True)
        pltpu.sync_copy(cstage, cgrid_sp.at[jown + _PAD])
        plsc.subcore_barrier()
        pltpu.sync_copy(cgrid_sp, cgridv)

        tot = jnp.zeros((_L,), jnp.int32)
        pre = jnp.zeros((_L,), jnp.int32)
        jv = jnp.full((_L,), jown, jnp.int32)
        for j in range(_NW):
            row = cgridv[j + _PAD]
            tot = tot + row
            pre = pre + jnp.where(jnp.full((_L,), j, jnp.int32) < jv, row, 0)
        nb = (tot + (_BM - 1)) // _BM
        cumnb = plsc.cumsum(nb)
        aligned = (cumnb - nb) * _BM
        mystart = aligned + pre
        for e in range(_E):
            run[e] = mystart[e]

        for pg in range(2 * PG):
            if pg < PG:
                ev = i1v[pl.ds(pg * _L, _L)]
            else:
                ev = i2v[pl.ds((pg - PG) * _L, _L)]
            slot = jnp.zeros((_L,), jnp.int32)
            for e in range(_E):
                m = ev == e
                mi = m.astype(jnp.int32)
                cs = plsc.cumsum(mi)
                r = run[e]
                slot = jnp.where(
                    m, jnp.full((_L,), r, jnp.int32) + cs - 1, slot)
                run[e] = r + jnp.sum(mi)
            slots[pl.ds(pg * _L, _L)] = slot

        t0 = jown * TPW
        pltpu.sync_copy(w1v, w1_hbm.at[pl.ds(t0, TPW)])
        pltpu.sync_copy(w2v, w2_hbm.at[pl.ds(t0, TPW)])
        pltpu.sync_copy(slots.at[pl.ds(0, TPW)], s1_hbm.at[pl.ds(t0, TPW)])
        pltpu.sync_copy(slots.at[pl.ds(TPW, TPW)], s2_hbm.at[pl.ds(t0, TPW)])

        for ch in range(TPW // XCH):
            base = ch * XCH
            pltpu.sync_copy(x_hbm.at[pl.ds(t0 + base, XCH)], xbuf)
            for q in range(XCH // _L):
                sl = pl.ds(q * _L, _L)
                idx1[sl] = jnp.clip(slots[pl.ds(base + q * _L, _L)], 0, N - 1)
                idx2[sl] = jnp.clip(
                    slots[pl.ds(TPW + base + q * _L, _L)], 0, N - 1)
            cp1 = pltpu.async_copy(xbuf, xs_hbm.at[idx1], sem1)
            cp2 = pltpu.async_copy(xbuf, xs_hbm.at[idx2], sem2)
            cp1.wait()
            cp2.wait()

        @pl.when(jown == 0)
        def _():
            cnb = jnp.int32(0)
            accs = [jnp.zeros((_L,), jnp.int32) for _ in range(GP // _L)]
            for e in range(_E):
                cnb = cnb + (tot[e] + (_BM - 1)) // _BM
                cnbv = jnp.full((_L,), cnb, jnp.int32)
                for v in range(GP // _L):
                    gvec = iota + v * _L
                    accs[v] = accs[v] + jnp.where(gvec >= cnbv, 1, 0)
            for v in range(GP // _L):
                bev[pl.ds(v * _L, _L)] = accs[v]
            pltpu.sync_copy(bev, be_hbm)

    return route


def _ffn_body(be_ref, xs_ref, gw_ref, uw_ref, dw_ref, o_ref):
    g = pl.program_id(0)
    be = be_ref[g]

    @pl.when(be < _E)
    def _():
        x = xs_ref[...].astype(jnp.bfloat16)
        gm = lax.dot_general(x, gw_ref[0], (((1,), (1,)), ((), ())),
                             preferred_element_type=jnp.float32)
        um = lax.dot_general(x, uw_ref[0], (((1,), (1,)), ((), ())),
                             preferred_element_type=jnp.float32)
        h = (gm * jax.nn.sigmoid(gm) * um).astype(jnp.bfloat16)
        o_ref[...] = lax.dot_general(h, dw_ref[0], (((1,), (1,)), ((), ())),
                                     preferred_element_type=jnp.float32)


def _grouped_ffn(be, xs, gw, uw, dw):
    N, D = xs.shape
    FF = gw.shape[1]
    G = N // _BM

    def _emap(g, be_r):
        return (jnp.minimum(be_r[g], _E - 1), 0, 0)

    return pl.pallas_call(
        _ffn_body,
        grid_spec=pltpu.PrefetchScalarGridSpec(
            num_scalar_prefetch=1,
            grid=(G,),
            in_specs=[
                pl.BlockSpec((_BM, D), lambda g, be_r: (g, 0)),
                pl.BlockSpec((1, FF, D), _emap),
                pl.BlockSpec((1, FF, D), _emap),
                pl.BlockSpec((1, D, FF), _emap),
            ],
            out_specs=pl.BlockSpec((_BM, D), lambda g, be_r: (g, 0)),
        ),
        out_shape=jax.ShapeDtypeStruct((N, D), jnp.float32),
        compiler_params=pltpu.CompilerParams(
            dimension_semantics=("arbitrary",)),
    )(be, xs, gw, uw, dw)


def _make_combine(T, D):
    TPW = T // _NW
    CCH = 32
    N = T * _K + _E * _BM

    mesh = plsc.VectorSubcoreMesh(core_axis_name="c", subcore_axis_name="s")

    @functools.partial(
        pl.kernel,
        out_type=jax.ShapeDtypeStruct((T, D), jnp.float32),
        mesh=mesh,
        scratch_types=[
            pltpu.VMEM((CCH,), jnp.int32),
            pltpu.VMEM((CCH,), jnp.int32),
            pltpu.VMEM((CCH,), jnp.float32),
            pltpu.VMEM((CCH,), jnp.float32),
            pltpu.VMEM((CCH, D), jnp.float32),
            pltpu.VMEM((CCH, D), jnp.float32),
            pltpu.VMEM((CCH, D), jnp.float32),
            pltpu.SemaphoreType.DMA,
            pltpu.SemaphoreType.DMA,
        ],
        compiler_params=pltpu.CompilerParams(needs_layout_passes=False),
    )
    def combine(o_hbm, s1_hbm, s2_hbm, w1_hbm, w2_hbm, fin_hbm,
                idx1, idx2, wa, wb, buf1, buf2, obuf, sem1, sem2):
        c = lax.axis_index("c")
        s = lax.axis_index("s")
        t0 = (c * _NS + s) * TPW
        for ch in range(TPW // CCH):
            tb = t0 + ch * CCH
            pltpu.sync_copy(s1_hbm.at[pl.ds(tb, CCH)], idx1)
            pltpu.sync_copy(s2_hbm.at[pl.ds(tb, CCH)], idx2)
            pltpu.sync_copy(w1_hbm.at[pl.ds(tb, CCH)], wa)
            pltpu.sync_copy(w2_hbm.at[pl.ds(tb, CCH)], wb)
            for q in range(CCH // _L):
                sl = pl.ds(q * _L, _L)
                idx1[sl] = jnp.clip(idx1[sl], 0, N - 1)
                idx2[sl] = jnp.clip(idx2[sl], 0, N - 1)
            cp1 = pltpu.async_copy(o_hbm.at[idx1], buf1, sem1)
            cp2 = pltpu.async_copy(o_hbm.at[idx2], buf2, sem2)
            cp1.wait()
            cp2.wait()

            def body(t, _):
                tv = jnp.full((_L,), t, jnp.int32)
                w1vec = plsc.load_gather(wa, [tv])
                w2vec = plsc.load_gather(wb, [tv])
                for v in range(D // _L):
                    sl = pl.ds(v * _L, _L)
                    obuf[t, sl] = buf1[t, sl] * w1vec + buf2[t, sl] * w2vec
                return 0

            lax.fori_loop(0, CCH, body, 0)
            pltpu.sync_copy(obuf, fin_hbm.at[pl.ds(tb, CCH)])

    return combine


def kernel(hidden_states, gate_w, gate_proj_w, up_proj_w, down_proj_w):
    B, S, D = hidden_states.shape
    T = B * S
    x = hidden_states.reshape(T, D)
    logits = _router_logits(x, gate_w)
    xs, s1, s2, w1, w2, be = _make_route_dispatch(T, D)(logits, x)
    o = _grouped_ffn(be, xs,
                     gate_proj_w.astype(jnp.bfloat16),
                     up_proj_w.astype(jnp.bfloat16),
                     down_proj_w.astype(jnp.bfloat16))
    fin = _make_combine(T, D)(o, s1, s2, w1, w2)
    return fin.reshape(B, S, D), logits


# speedup vs baseline: 1.1206x; 1.1206x over previous
import functools

import jax
import jax.numpy as jnp
from jax import lax
from jax.experimental import pallas as pl
from jax.experimental.pallas import tpu as pltpu
from jax.experimental.pallas import tpu_sc as plsc

_E = 8
_K = 2
_BM = 128
_FFC = 512
_L = 16
_NC = 2
_NS = 16
_NW = _NC * _NS
_PAD = 8


def _logits_body(x_ref, gw_ref, out_ref):
    out_ref[...] = lax.dot_general(
        x_ref[...], gw_ref[...], (((1,), (1,)), ((), ())),
        preferred_element_type=jnp.float32)


def _router_logits(x, gate_w):
    T, D = x.shape
    bt = 1024
    return pl.pallas_call(
        _logits_body,
        grid=(T // bt,),
        in_specs=[pl.BlockSpec((bt, D), lambda i: (i, 0)),
                  pl.BlockSpec((_E, D), lambda i: (0, 0))],
        out_specs=pl.BlockSpec((bt, _E), lambda i: (i, 0)),
        out_shape=jax.ShapeDtypeStruct((T, _E), jnp.float32),
    )(x, gate_w)


def _make_route_dispatch(T, D):
    TPW = T // _NW
    PG = TPW // _L
    N = T * _K + _E * _BM
    G = N // _BM
    GP = ((G + _L - 1) // _L) * _L
    XCH = 64

    mesh = plsc.VectorSubcoreMesh(core_axis_name="c", subcore_axis_name="s")

    @functools.partial(
        pl.kernel,
        out_type=[
            jax.ShapeDtypeStruct((N, D), jnp.float32),
            jax.ShapeDtypeStruct((T,), jnp.int32),
            jax.ShapeDtypeStruct((T,), jnp.int32),
            jax.ShapeDtypeStruct((T,), jnp.float32),
            jax.ShapeDtypeStruct((T,), jnp.float32),
            jax.ShapeDtypeStruct((GP,), jnp.int32),
        ],
        mesh=mesh,
        scratch_types=[
            pltpu.VMEM((TPW, _E), jnp.float32),
            pltpu.VMEM((TPW,), jnp.int32),
            pltpu.VMEM((TPW,), jnp.int32),
            pltpu.VMEM((TPW,), jnp.float32),
            pltpu.VMEM((TPW,), jnp.float32),
            pltpu.VMEM((2 * TPW,), jnp.int32),
            pltpu.VMEM((_L,), jnp.int32),
            pltpu.VMEM((_NW + _PAD, _L), jnp.int32),
            pltpu.VMEM((XCH, D), jnp.float32),
            pltpu.VMEM((XCH,), jnp.int32),
            pltpu.VMEM((XCH,), jnp.int32),
            pltpu.VMEM((GP,), jnp.int32),
            pltpu.SMEM((_E,), jnp.int32),
            pltpu.VMEM_SHARED((_NW + _PAD, _L), jnp.int32),
            pltpu.SemaphoreType.DMA,
            pltpu.SemaphoreType.DMA,
        ],
        compiler_params=pltpu.CompilerParams(needs_layout_passes=False),
    )
    def route(logits_hbm, x_hbm, xs_hbm, s1_hbm, s2_hbm, w1_hbm, w2_hbm,
              be_hbm, lgbuf, i1v, i2v, w1v, w2v, slots, cstage, cgridv,
              xbuf, idx1, idx2, bev, run, cgrid_sp, sem1, sem2):
        c = lax.axis_index("c")
        s = lax.axis_index("s")
        jown = c * _NS + s
        jalt = (1 - c) * _NS + s
        iota = lax.iota(jnp.int32, _L)

        def top2_counts(store):
            cnt = jnp.zeros((_L,), jnp.int32)
            for g in range(PG):
                rows = g * _L + iota
                vals = [
                    plsc.load_gather(
                        lgbuf, [rows, jnp.full((_L,), e, jnp.int32)])
                    for e in range(_E)
                ]
                m1 = vals[0]
                e1 = jnp.zeros((_L,), jnp.int32)
                for e in range(1, _E):
                    gt = vals[e] > m1
                    m1 = jnp.where(gt, vals[e], m1)
                    e1 = jnp.where(gt, e, e1)
                m2 = jnp.full((_L,), -jnp.inf, jnp.float32)
                e2 = jnp.zeros((_L,), jnp.int32)
                for e in range(_E):
                    ve = jnp.where(e1 == e, -jnp.inf, vals[e])
                    gt = ve > m2
                    m2 = jnp.where(gt, ve, m2)
                    e2 = jnp.where(gt, e, e2)
                for e in range(_E):
                    pc = (plsc.all_reduce_population_count(e1 == e)
                          + plsc.all_reduce_population_count(e2 == e))
                    cnt = jnp.where(iota == e, cnt + pc, cnt)
                if store:
                    sl = pl.ds(g * _L, _L)
                    i1v[sl] = e1
                    i2v[sl] = e2
                    ww1 = 1.0 / (1.0 + jnp.exp(m2 - m1))
                    w1v[sl] = ww1
                    w2v[sl] = 1.0 - ww1
            return cnt

        pltpu.sync_copy(logits_hbm.at[pl.ds(jalt * TPW, TPW)], lgbuf)
        cstage[...] = top2_counts(False)
        pltpu.sync_copy(cstage, cgrid_sp.at[jalt + _PAD])
        pltpu.sync_copy(logits_hbm.at[pl.ds(jown * TPW, TPW)], lgbuf)
        cstage[...] = top2_counts(True)
        pltpu.sync_copy(cstage, cgrid_sp.at[jown + _PAD])
        plsc.subcore_barrier()
        pltpu.sync_copy(cgrid_sp, cgridv)

        tot = jnp.zeros((_L,), jnp.int32)
        pre = jnp.zeros((_L,), jnp.int32)
        jv = jnp.full((_L,), jown, jnp.int32)
        for j in range(_NW):
            row = cgridv[j + _PAD]
            tot = tot + row
            pre = pre + jnp.where(jnp.full((_L,), j, jnp.int32) < jv, row, 0)
        nb = (tot + (_BM - 1)) // _BM
        cumnb = plsc.cumsum(nb)
        aligned = (cumnb - nb) * _BM
        mystart = aligned + pre
        for e in range(_E):
            run[e] = mystart[e]

        for pg in range(2 * PG):
            if pg < PG:
                ev = i1v[pl.ds(pg * _L, _L)]
            else:
                ev = i2v[pl.ds((pg - PG) * _L, _L)]
            slot = jnp.zeros((_L,), jnp.int32)
            for e in range(_E):
                m = ev == e
                mi = m.astype(jnp.int32)
                cs = plsc.cumsum(mi)
                r = run[e]
                slot = jnp.where(
                    m, jnp.full((_L,), r, jnp.int32) + cs - 1, slot)
                run[e] = r + jnp.sum(mi)
            slots[pl.ds(pg * _L, _L)] = slot

        t0 = jown * TPW
        pltpu.sync_copy(w1v, w1_hbm.at[pl.ds(t0, TPW)])
        pltpu.sync_copy(w2v, w2_hbm.at[pl.ds(t0, TPW)])
        pltpu.sync_copy(slots.at[pl.ds(0, TPW)], s1_hbm.at[pl.ds(t0, TPW)])
        pltpu.sync_copy(slots.at[pl.ds(TPW, TPW)], s2_hbm.at[pl.ds(t0, TPW)])

        for ch in range(TPW // XCH):
            base = ch * XCH
            pltpu.sync_copy(x_hbm.at[pl.ds(t0 + base, XCH)], xbuf)
            for q in range(XCH // _L):
                sl = pl.ds(q * _L, _L)
                idx1[sl] = jnp.clip(slots[pl.ds(base + q * _L, _L)], 0, N - 1)
                idx2[sl] = jnp.clip(
                    slots[pl.ds(TPW + base + q * _L, _L)], 0, N - 1)
            cp1 = pltpu.async_copy(xbuf, xs_hbm.at[idx1], sem1)
            cp2 = pltpu.async_copy(xbuf, xs_hbm.at[idx2], sem2)
            cp1.wait()
            cp2.wait()

        @pl.when(jown == 0)
        def _():
            cnb = jnp.int32(0)
            accs = [jnp.zeros((_L,), jnp.int32) for _ in range(GP // _L)]
            for e in range(_E):
                cnb = cnb + (tot[e] + (_BM - 1)) // _BM
                cnbv = jnp.full((_L,), cnb, jnp.int32)
                for v in range(GP // _L):
                    gvec = iota + v * _L
                    accs[v] = accs[v] + jnp.where(gvec >= cnbv, 1, 0)
            for v in range(GP // _L):
                bev[pl.ds(v * _L, _L)] = accs[v]
            pltpu.sync_copy(bev, be_hbm)

    return route


def _ffn_body(be_ref, xs_ref, gw_ref, uw_ref, dw_ref, o_ref):
    g = pl.program_id(0)
    be = be_ref[g]

    @pl.when(be < _E)
    def _():
        x = xs_ref[...]
        gm = lax.dot_general(x, gw_ref[0], (((1,), (1,)), ((), ())),
                             preferred_element_type=jnp.float32,
                             precision=lax.Precision.DEFAULT)
        um = lax.dot_general(x, uw_ref[0], (((1,), (1,)), ((), ())),
                             preferred_element_type=jnp.float32,
                             precision=lax.Precision.DEFAULT)
        h = gm * jax.nn.sigmoid(gm) * um
        o_ref[...] = lax.dot_general(h, dw_ref[0], (((1,), (1,)), ((), ())),
                                     preferred_element_type=jnp.float32,
                                     precision=lax.Precision.DEFAULT)


def _grouped_ffn(be, xs, gw, uw, dw):
    N, D = xs.shape
    FF = gw.shape[1]
    G = N // _BM

    def _emap(g, be_r):
        return (jnp.minimum(be_r[g], _E - 1), 0, 0)

    return pl.pallas_call(
        _ffn_body,
        grid_spec=pltpu.PrefetchScalarGridSpec(
            num_scalar_prefetch=1,
            grid=(G,),
            in_specs=[
                pl.BlockSpec((_BM, D), lambda g, be_r: (g, 0)),
                pl.BlockSpec((1, FF, D), _emap),
                pl.BlockSpec((1, FF, D), _emap),
                pl.BlockSpec((1, D, FF), _emap),
            ],
            out_specs=pl.BlockSpec((_BM, D), lambda g, be_r: (g, 0)),
        ),
        out_shape=jax.ShapeDtypeStruct((N, D), jnp.float32),
        compiler_params=pltpu.CompilerParams(
            dimension_semantics=("arbitrary",)),
    )(be, xs, gw, uw, dw)


def _make_combine(T, D):
    TPW = T // _NW
    CCH = 32
    N = T * _K + _E * _BM

    mesh = plsc.VectorSubcoreMesh(core_axis_name="c", subcore_axis_name="s")

    @functools.partial(
        pl.kernel,
        out_type=jax.ShapeDtypeStruct((T, D), jnp.float32),
        mesh=mesh,
        scratch_types=[
            pltpu.VMEM((CCH,), jnp.int32),
            pltpu.VMEM((CCH,), jnp.int32),
            pltpu.VMEM((CCH,), jnp.float32),
            pltpu.VMEM((CCH,), jnp.float32),
            pltpu.VMEM((CCH, D), jnp.float32),
            pltpu.VMEM((CCH, D), jnp.float32),
            pltpu.VMEM((CCH, D), jnp.float32),
            pltpu.SemaphoreType.DMA,
            pltpu.SemaphoreType.DMA,
        ],
        compiler_params=pltpu.CompilerParams(needs_layout_passes=False),
    )
    def combine(o_hbm, s1_hbm, s2_hbm, w1_hbm, w2_hbm, fin_hbm,
                idx1, idx2, wa, wb, buf1, buf2, obuf, sem1, sem2):
        c = lax.axis_index("c")
        s = lax.axis_index("s")
        t0 = (c * _NS + s) * TPW
        for ch in range(TPW // CCH):
            tb = t0 + ch * CCH
            pltpu.sync_copy(s1_hbm.at[pl.ds(tb, CCH)], idx1)
            pltpu.sync_copy(s2_hbm.at[pl.ds(tb, CCH)], idx2)
            pltpu.sync_copy(w1_hbm.at[pl.ds(tb, CCH)], wa)
            pltpu.sync_copy(w2_hbm.at[pl.ds(tb, CCH)], wb)
            for q in range(CCH // _L):
                sl = pl.ds(q * _L, _L)
                idx1[sl] = jnp.clip(idx1[sl], 0, N - 1)
                idx2[sl] = jnp.clip(idx2[sl], 0, N - 1)
            cp1 = pltpu.async_copy(o_hbm.at[idx1], buf1, sem1)
            cp2 = pltpu.async_copy(o_hbm.at[idx2], buf2, sem2)
            cp1.wait()
            cp2.wait()

            def body(t, _):
                tv = jnp.full((_L,), t, jnp.int32)
                w1vec = plsc.load_gather(wa, [tv])
                w2vec = plsc.load_gather(wb, [tv])
                for v in range(D // _L):
                    sl = pl.ds(v * _L, _L)
                    obuf[t, sl] = buf1[t, sl] * w1vec + buf2[t, sl] * w2vec
                return 0

            lax.fori_loop(0, CCH, body, 0)
            pltpu.sync_copy(obuf, fin_hbm.at[pl.ds(tb, CCH)])

    return combine


def kernel(hidden_states, gate_w, gate_proj_w, up_proj_w, down_proj_w):
    B, S, D = hidden_states.shape
    T = B * S
    x = hidden_states.reshape(T, D)
    logits = _router_logits(x, gate_w)
    xs, s1, s2, w1, w2, be = _make_route_dispatch(T, D)(logits, x)
    o = _grouped_ffn(be, xs, gate_proj_w, up_proj_w, down_proj_w)
    fin = _make_combine(T, D)(o, s1, s2, w1, w2)
    return fin.reshape(B, S, D), logits


# speedup vs baseline: 1.8016x; 1.6077x over previous
import functools

import jax
import jax.numpy as jnp
from jax import lax
from jax.experimental import pallas as pl
from jax.experimental.pallas import tpu as pltpu
from jax.experimental.pallas import tpu_sc as plsc

_E = 8
_K = 2
_BM = 256
_FFC = 512
_L = 16
_NC = 2
_NS = 16
_NW = _NC * _NS
_PAD = 8


def _logits_body(x_ref, gw_ref, out_ref):
    out_ref[...] = lax.dot_general(
        x_ref[...], gw_ref[...], (((1,), (1,)), ((), ())),
        preferred_element_type=jnp.float32)


def _router_logits(x, gate_w):
    T, D = x.shape
    bt = 1024
    return pl.pallas_call(
        _logits_body,
        grid=(T // bt,),
        in_specs=[pl.BlockSpec((bt, D), lambda i: (i, 0)),
                  pl.BlockSpec((_E, D), lambda i: (0, 0))],
        out_specs=pl.BlockSpec((bt, _E), lambda i: (i, 0)),
        out_shape=jax.ShapeDtypeStruct((T, _E), jnp.float32),
    )(x, gate_w)


def _make_route_dispatch(T, D):
    TPW = T // _NW
    PG = TPW // _L
    N = T * _K + _E * _BM
    G = N // _BM
    GP = ((G + _L - 1) // _L) * _L
    XCH = 64

    mesh = plsc.VectorSubcoreMesh(core_axis_name="c", subcore_axis_name="s")

    @functools.partial(
        pl.kernel,
        out_type=[
            jax.ShapeDtypeStruct((N, D), jnp.float32),
            jax.ShapeDtypeStruct((T,), jnp.int32),
            jax.ShapeDtypeStruct((T,), jnp.int32),
            jax.ShapeDtypeStruct((T,), jnp.float32),
            jax.ShapeDtypeStruct((T,), jnp.float32),
            jax.ShapeDtypeStruct((GP,), jnp.int32),
        ],
        mesh=mesh,
        scratch_types=[
            pltpu.VMEM((TPW, _E), jnp.float32),
            pltpu.VMEM((TPW,), jnp.int32),
            pltpu.VMEM((TPW,), jnp.int32),
            pltpu.VMEM((TPW,), jnp.float32),
            pltpu.VMEM((TPW,), jnp.float32),
            pltpu.VMEM((2 * TPW,), jnp.int32),
            pltpu.VMEM((_L,), jnp.int32),
            pltpu.VMEM((_NW + _PAD, _L), jnp.int32),
            pltpu.VMEM((XCH, D), jnp.float32),
            pltpu.VMEM((XCH,), jnp.int32),
            pltpu.VMEM((XCH,), jnp.int32),
            pltpu.VMEM((GP,), jnp.int32),
            pltpu.SMEM((_E,), jnp.int32),
            pltpu.VMEM_SHARED((_NW + _PAD, _L), jnp.int32),
            pltpu.SemaphoreType.DMA,
            pltpu.SemaphoreType.DMA,
        ],
        compiler_params=pltpu.CompilerParams(needs_layout_passes=False),
    )
    def route(logits_hbm, x_hbm, xs_hbm, s1_hbm, s2_hbm, w1_hbm, w2_hbm,
              be_hbm, lgbuf, i1v, i2v, w1v, w2v, slots, cstage, cgridv,
              xbuf, idx1, idx2, bev, run, cgrid_sp, sem1, sem2):
        c = lax.axis_index("c")
        s = lax.axis_index("s")
        jown = c * _NS + s
        jalt = (1 - c) * _NS + s
        iota = lax.iota(jnp.int32, _L)

        def top2_counts(store):
            cnt = jnp.zeros((_L,), jnp.int32)
            for g in range(PG):
                rows = g * _L + iota
                vals = [
                    plsc.load_gather(
                        lgbuf, [rows, jnp.full((_L,), e, jnp.int32)])
                    for e in range(_E)
                ]
                m1 = vals[0]
                e1 = jnp.zeros((_L,), jnp.int32)
                for e in range(1, _E):
                    gt = vals[e] > m1
                    m1 = jnp.where(gt, vals[e], m1)
                    e1 = jnp.where(gt, e, e1)
                m2 = jnp.full((_L,), -jnp.inf, jnp.float32)
                e2 = jnp.zeros((_L,), jnp.int32)
                for e in range(_E):
                    ve = jnp.where(e1 == e, -jnp.inf, vals[e])
                    gt = ve > m2
                    m2 = jnp.where(gt, ve, m2)
                    e2 = jnp.where(gt, e, e2)
                for e in range(_E):
                    pc = (plsc.all_reduce_population_count(e1 == e)
                          + plsc.all_reduce_population_count(e2 == e))
                    cnt = jnp.where(iota == e, cnt + pc, cnt)
                if store:
                    sl = pl.ds(g * _L, _L)
                    i1v[sl] = e1
                    i2v[sl] = e2
                    ww1 = 1.0 / (1.0 + jnp.exp(m2 - m1))
                    w1v[sl] = ww1
                    w2v[sl] = 1.0 - ww1
            return cnt

        pltpu.sync_copy(logits_hbm.at[pl.ds(jalt * TPW, TPW)], lgbuf)
        cstage[...] = top2_counts(False)
        pltpu.sync_copy(cstage, cgrid_sp.at[jalt + _PAD])
        pltpu.sync_copy(logits_hbm.at[pl.ds(jown * TPW, TPW)], lgbuf)
        cstage[...] = top2_counts(True)
        pltpu.sync_copy(cstage, cgrid_sp.at[jown + _PAD])
        plsc.subcore_barrier()
        pltpu.sync_copy(cgrid_sp, cgridv)

        tot = jnp.zeros((_L,), jnp.int32)
        pre = jnp.zeros((_L,), jnp.int32)
        jv = jnp.full((_L,), jown, jnp.int32)
        for j in range(_NW):
            row = cgridv[j + _PAD]
            tot = tot + row
            pre = pre + jnp.where(jnp.full((_L,), j, jnp.int32) < jv, row, 0)
        nb = (tot + (_BM - 1)) // _BM
        cumnb = plsc.cumsum(nb)
        aligned = (cumnb - nb) * _BM
        mystart = aligned + pre
        for e in range(_E):
            run[e] = mystart[e]

        for pg in range(2 * PG):
            if pg < PG:
                ev = i1v[pl.ds(pg * _L, _L)]
            else:
                ev = i2v[pl.ds((pg - PG) * _L, _L)]
            slot = jnp.zeros((_L,), jnp.int32)
            for e in range(_E):
                m = ev == e
                mi = m.astype(jnp.int32)
                cs = plsc.cumsum(mi)
                r = run[e]
                slot = jnp.where(
                    m, jnp.full((_L,), r, jnp.int32) + cs - 1, slot)
                run[e] = r + jnp.sum(mi)
            slots[pl.ds(pg * _L, _L)] = slot

        t0 = jown * TPW
        pltpu.sync_copy(w1v, w1_hbm.at[pl.ds(t0, TPW)])
        pltpu.sync_copy(w2v, w2_hbm.at[pl.ds(t0, TPW)])
        pltpu.sync_copy(slots.at[pl.ds(0, TPW)], s1_hbm.at[pl.ds(t0, TPW)])
        pltpu.sync_copy(slots.at[pl.ds(TPW, TPW)], s2_hbm.at[pl.ds(t0, TPW)])

        for ch in range(TPW // XCH):
            base = ch * XCH
            pltpu.sync_copy(x_hbm.at[pl.ds(t0 + base, XCH)], xbuf)
            for q in range(XCH // _L):
                sl = pl.ds(q * _L, _L)
                idx1[sl] = jnp.clip(slots[pl.ds(base + q * _L, _L)], 0, N - 1)
                idx2[sl] = jnp.clip(
                    slots[pl.ds(TPW + base + q * _L, _L)], 0, N - 1)
            cp1 = pltpu.async_copy(xbuf, xs_hbm.at[idx1], sem1)
            cp2 = pltpu.async_copy(xbuf, xs_hbm.at[idx2], sem2)
            cp1.wait()
            cp2.wait()

        @pl.when(jown == 0)
        def _():
            cnb = jnp.int32(0)
            accs = [jnp.zeros((_L,), jnp.int32) for _ in range(GP // _L)]
            for e in range(_E):
                cnb = cnb + (tot[e] + (_BM - 1)) // _BM
                cnbv = jnp.full((_L,), cnb, jnp.int32)
                for v in range(GP // _L):
                    gvec = iota + v * _L
                    accs[v] = accs[v] + jnp.where(gvec >= cnbv, 1, 0)
            for v in range(GP // _L):
                bev[pl.ds(v * _L, _L)] = accs[v]
            pltpu.sync_copy(bev, be_hbm)

    return route


def _ffn_body(be_ref, xs_ref, gw_ref, uw_ref, dw_ref, o_ref):
    g = pl.program_id(0)
    be = be_ref[g]

    @pl.when(be < _E)
    def _():
        x = xs_ref[...]
        gm = lax.dot_general(x, gw_ref[0], (((1,), (1,)), ((), ())),
                             preferred_element_type=jnp.float32,
                             precision=lax.Precision.DEFAULT)
        um = lax.dot_general(x, uw_ref[0], (((1,), (1,)), ((), ())),
                             preferred_element_type=jnp.float32,
                             precision=lax.Precision.DEFAULT)
        h = gm * jax.nn.sigmoid(gm) * um
        o_ref[...] = lax.dot_general(h, dw_ref[0], (((1,), (1,)), ((), ())),
                                     preferred_element_type=jnp.float32,
                                     precision=lax.Precision.DEFAULT)


def _grouped_ffn(be, xs, gw, uw, dw):
    N, D = xs.shape
    FF = gw.shape[1]
    G = N // _BM

    def _emap(g, be_r):
        return (jnp.minimum(be_r[g], _E - 1), 0, 0)

    return pl.pallas_call(
        _ffn_body,
        grid_spec=pltpu.PrefetchScalarGridSpec(
            num_scalar_prefetch=1,
            grid=(G,),
            in_specs=[
                pl.BlockSpec((_BM, D), lambda g, be_r: (g, 0)),
                pl.BlockSpec((1, FF, D), _emap),
                pl.BlockSpec((1, FF, D), _emap),
                pl.BlockSpec((1, D, FF), _emap),
            ],
            out_specs=pl.BlockSpec((_BM, D), lambda g, be_r: (g, 0)),
        ),
        out_shape=jax.ShapeDtypeStruct((N, D), jnp.float32),
        compiler_params=pltpu.CompilerParams(
            dimension_semantics=("arbitrary",)),
    )(be, xs, gw, uw, dw)


def _make_combine(T, D):
    TPW = T // _NW
    CCH = 32
    N = T * _K + _E * _BM

    mesh = plsc.VectorSubcoreMesh(core_axis_name="c", subcore_axis_name="s")

    @functools.partial(
        pl.kernel,
        out_type=jax.ShapeDtypeStruct((T, D), jnp.float32),
        mesh=mesh,
        scratch_types=[
            pltpu.VMEM((CCH,), jnp.int32),
            pltpu.VMEM((CCH,), jnp.int32),
            pltpu.VMEM((CCH,), jnp.float32),
            pltpu.VMEM((CCH,), jnp.float32),
            pltpu.VMEM((CCH, D), jnp.float32),
            pltpu.VMEM((CCH, D), jnp.float32),
            pltpu.VMEM((CCH, D), jnp.float32),
            pltpu.SemaphoreType.DMA,
            pltpu.SemaphoreType.DMA,
        ],
        compiler_params=pltpu.CompilerParams(needs_layout_passes=False),
    )
    def combine(o_hbm, s1_hbm, s2_hbm, w1_hbm, w2_hbm, fin_hbm,
                idx1, idx2, wa, wb, buf1, buf2, obuf, sem1, sem2):
        c = lax.axis_index("c")
        s = lax.axis_index("s")
        t0 = (c * _NS + s) * TPW
        for ch in range(TPW // CCH):
            tb = t0 + ch * CCH
            pltpu.sync_copy(s1_hbm.at[pl.ds(tb, CCH)], idx1)
            pltpu.sync_copy(s2_hbm.at[pl.ds(tb, CCH)], idx2)
            pltpu.sync_copy(w1_hbm.at[pl.ds(tb, CCH)], wa)
            pltpu.sync_copy(w2_hbm.at[pl.ds(tb, CCH)], wb)
            for q in range(CCH // _L):
                sl = pl.ds(q * _L, _L)
                idx1[sl] = jnp.clip(idx1[sl], 0, N - 1)
                idx2[sl] = jnp.clip(idx2[sl], 0, N - 1)
            cp1 = pltpu.async_copy(o_hbm.at[idx1], buf1, sem1)
            cp2 = pltpu.async_copy(o_hbm.at[idx2], buf2, sem2)
            cp1.wait()
            cp2.wait()

            def body(t, _):
                tv = jnp.full((_L,), t, jnp.int32)
                w1vec = plsc.load_gather(wa, [tv])
                w2vec = plsc.load_gather(wb, [tv])
                for v in range(D // _L):
                    sl = pl.ds(v * _L, _L)
                    obuf[t, sl] = buf1[t, sl] * w1vec + buf2[t, sl] * w2vec
                return 0

            lax.fori_loop(0, CCH, body, 0)
            pltpu.sync_copy(obuf, fin_hbm.at[pl.ds(tb, CCH)])

    return combine


def kernel(hidden_states, gate_w, gate_proj_w, up_proj_w, down_proj_w):
    B, S, D = hidden_states.shape
    T = B * S
    x = hidden_states.reshape(T, D)
    logits = _router_logits(x, gate_w)
    xs, s1, s2, w1, w2, be = _make_route_dispatch(T, D)(logits, x)
    o = _grouped_ffn(be, xs, gate_proj_w, up_proj_w, down_proj_w)
    fin = _make_combine(T, D)(o, s1, s2, w1, w2)
    return fin.reshape(B, S, D), logits
